# baseline (speedup 1.0000x reference)
import jax
import jax.numpy as jnp
from jax.experimental import pallas as pl
from jax.experimental.pallas import tpu as pltpu

_E = 64
_TOPK = 2
_H = 768
_I = 768
_T = 2048
_R = _T * _TOPK
_BM = 128
_MAXPAD = _R + _E * _BM
_NBLK = _MAXPAD // _BM
_LA = 3
_NRING = 5
_RXN = _NBLK + _LA + 1

_INTERPRET = False


def _ffn_block(info_ref, runid_ref, fb_ref, runx_ref,
               x_ref, w1_hbm, b1_ref, sel_ref, w2_hbm, b2_ref, wp_ref,
               out_ref, w1buf, w2buf, sem1, sem2):
    i = pl.program_id(0)
    nused = info_ref[_NBLK]
    nruns = runx_ref[_RXN - 1]

    def issue(run, slot):
        e = runx_ref[run]
        pltpu.make_async_copy(w1_hbm.at[e], w1buf.at[slot], sem1.at[slot]).start()
        pltpu.make_async_copy(w2_hbm.at[e], w2buf.at[slot], sem2.at[slot]).start()

    @pl.when(i == 0)
    def _():
        for k in range(_LA):
            @pl.when(k < nruns)
            def _():
                issue(k, k)

    @pl.when((fb_ref[i] == 1) & (i < nused))
    def _():
        r = runid_ref[i]

        @pl.when(r + _LA < nruns)
        def _():
            issue(r + _LA, jax.lax.rem(r + _LA, _NRING))

        slot = jax.lax.rem(r, _NRING)
        e = runx_ref[r]
        pltpu.make_async_copy(w1_hbm.at[e], w1buf.at[slot], sem1.at[slot]).wait()
        pltpu.make_async_copy(w2_hbm.at[e], w2buf.at[slot], sem2.at[slot]).wait()

    @pl.when(i < nused)
    def _():
        slot = jax.lax.rem(runid_ref[i], _NRING)
        x = x_ref[...]
        w1e = w1buf[slot]
        h = jax.lax.dot_general(x, w1e, (((1,), (1,)), ((), ())),
                                preferred_element_type=jnp.float32)
        h = h + b1_ref[0, 0]
        hr = pltpu.roll(h, 2 * _I - 1, 1)
        p = h * jax.nn.sigmoid(1.702 * h) * (hr + 1.0)
        s = jax.lax.dot_general(p, sel_ref[...], (((1,), (0,)), ((), ())),
                                preferred_element_type=jnp.float32)
        y = jax.lax.dot_general(s, w2buf[slot], (((1,), (1,)), ((), ())),
                                preferred_element_type=jnp.float32)
        y = y + b2_ref[0, 0]
        out_ref[...] = y * wp_ref[0, 0][:, None]


def kernel(hidden_states, gate_w, gate_b, w1, b1, w2, b2):
    bsz, seq, hd = hidden_states.shape
    x2 = hidden_states.reshape(-1, hd)

    logits = x2 @ gate_w.T + gate_b
    probs = jax.nn.softmax(logits, axis=-1)
    vals, idx = jax.lax.top_k(probs, _TOPK)
    vals = vals / jnp.sum(vals, axis=-1, keepdims=True)

    e_flat = idx.reshape(-1).astype(jnp.int32)
    v_flat = vals.reshape(-1)
    order = jnp.argsort(e_flat, stable=True)
    rank = jnp.zeros((_R,), jnp.int32).at[order].set(
        jnp.arange(_R, dtype=jnp.int32))
    counts = jnp.bincount(e_flat, length=_E).astype(jnp.int32)
    pcounts = ((counts + _BM - 1) // _BM) * _BM
    pc_cum = jnp.cumsum(pcounts).astype(jnp.int32)
    pstart = pc_cum - pcounts
    g_cum = jnp.cumsum(counts).astype(jnp.int32)
    gstart = g_cum - counts
    total_pad = pc_cum[-1]
    nused = (total_pad // _BM).astype(jnp.int32)
    pos = pstart[e_flat] + (rank - gstart[e_flat])
    src_tok = jnp.zeros((_MAXPAD,), jnp.int32).at[pos].set(
        jnp.arange(_R, dtype=jnp.int32) // _TOPK)
    w_pad = jnp.zeros((_MAXPAD,), jnp.float32).at[pos].set(v_flat)
    queries = (jnp.arange(_NBLK, dtype=jnp.int32) * _BM).astype(jnp.int32)
    be = jnp.searchsorted(pc_cum, queries, side="right").astype(jnp.int32)
    be_last = be[jnp.maximum(nused - 1, 0)]
    be = jnp.where(queries < total_pad, be, be_last)
    info = jnp.concatenate([be, nused[None]])
    fb = jnp.concatenate([jnp.ones((1,), jnp.int32),
                          (be[1:] != be[:-1]).astype(jnp.int32)])
    fb = fb * (queries < total_pad).astype(jnp.int32)
    runid = jnp.cumsum(fb).astype(jnp.int32) - 1
    nruns = jnp.sum(fb).astype(jnp.int32)
    runx = jnp.zeros((_RXN,), jnp.int32).at[runid].set(be)
    runx = runx.at[_RXN - 1].set(nruns)

    x_pad = x2[src_tok]

    b1r = b1.reshape(_E, 1, 2 * _I)
    b2r = b2.reshape(_E, 1, _H)
    wpr = w_pad.reshape(_NBLK, 1, _BM)
    sel = (jnp.arange(2 * _I, dtype=jnp.int32)[:, None]
           == 2 * jnp.arange(_I, dtype=jnp.int32)[None, :]).astype(jnp.float32)
    grid_spec = pltpu.PrefetchScalarGridSpec(
        num_scalar_prefetch=4,
        grid=(_NBLK,),
        in_specs=[
            pl.BlockSpec((_BM, _H), lambda i, *s: (i, 0)),
            pl.BlockSpec(memory_space=pltpu.MemorySpace.HBM),
            pl.BlockSpec((1, 1, 2 * _I), lambda i, *s: (s[0][i], 0, 0)),
            pl.BlockSpec((2 * _I, _I), lambda i, *s: (0, 0)),
            pl.BlockSpec(memory_space=pltpu.MemorySpace.HBM),
            pl.BlockSpec((1, 1, _H), lambda i, *s: (s[0][i], 0, 0)),
            pl.BlockSpec((1, 1, _BM), lambda i, *s: (i, 0, 0)),
        ],
        out_specs=pl.BlockSpec((_BM, _H), lambda i, *s: (i, 0)),
        scratch_shapes=[
            pltpu.VMEM((_NRING, 2 * _I, _H), jnp.float32),
            pltpu.VMEM((_NRING, _H, _I), jnp.float32),
            pltpu.SemaphoreType.DMA((_NRING,)),
            pltpu.SemaphoreType.DMA((_NRING,)),
        ],
    )
    y_pad = pl.pallas_call(
        _ffn_block,
        grid_spec=grid_spec,
        out_shape=jax.ShapeDtypeStruct((_MAXPAD, _H), jnp.float32),
        compiler_params=pltpu.CompilerParams(
            dimension_semantics=("arbitrary",)),
        interpret=_INTERPRET,
    )(info, runid, fb, runx, x_pad, w1, b1r, sel, w2, b2r, wpr)

    p2 = pos.reshape(_T, _TOPK)
    out2 = y_pad[p2[:, 0]] + y_pad[p2[:, 1]]
    return out2.reshape(bsz, seq, hd)


# speedup vs baseline: 1.0050x; 1.0050x over previous
import jax
import jax.numpy as jnp
from jax.experimental import pallas as pl
from jax.experimental.pallas import tpu as pltpu

_E = 64
_TOPK = 2
_H = 768
_I = 768
_T = 2048
_R = _T * _TOPK
_BM = 128
_MAXPAD = _R + _E * _BM
_NBLK = _MAXPAD // _BM
_LA = 3
_NRING = 5
_RXN = _NBLK + _LA + 1

_INTERPRET = False


def _ffn_block(info_ref, runid_ref, fb_ref, runx_ref,
               x_ref, w1_hbm, b1_ref, sel_ref, w2_hbm, b2_ref, wp_ref,
               out_ref, w1bufa, w1bufb, w2bufa, w2bufb,
               sem1a, sem1b, sem2a, sem2b):
    i = pl.program_id(0)
    nused = info_ref[_NBLK]
    nruns = runx_ref[_RXN - 1]

    def copies(run, slot):
        e = runx_ref[run]
        return (
            pltpu.make_async_copy(w1_hbm.at[e, pl.ds(0, _I)],
                                  w1bufa.at[slot], sem1a.at[slot]),
            pltpu.make_async_copy(w1_hbm.at[e, pl.ds(_I, _I)],
                                  w1bufb.at[slot], sem1b.at[slot]),
            pltpu.make_async_copy(w2_hbm.at[e, pl.ds(0, _H // 2)],
                                  w2bufa.at[slot], sem2a.at[slot]),
            pltpu.make_async_copy(w2_hbm.at[e, pl.ds(_H // 2, _H // 2)],
                                  w2bufb.at[slot], sem2b.at[slot]),
        )

    @pl.when(i == 0)
    def _():
        for k in range(_LA):
            @pl.when(k < nruns)
            def _():
                for c in copies(k, k):
                    c.start()

    @pl.when((fb_ref[i] == 1) & (i < nused))
    def _():
        r = runid_ref[i]

        @pl.when(r + _LA < nruns)
        def _():
            for c in copies(r + _LA, jax.lax.rem(r + _LA, _NRING)):
                c.start()

        for c in copies(r, jax.lax.rem(r, _NRING)):
            c.wait()

    @pl.when(i < nused)
    def _():
        slot = jax.lax.rem(runid_ref[i], _NRING)
        x = x_ref[...]
        b1 = b1_ref[0, 0]

        def shalf(wbuf, k):
            h = jax.lax.dot_general(x, wbuf[slot], (((1,), (1,)), ((), ())),
                                    preferred_element_type=jnp.float32)
            h = h + jax.lax.slice_in_dim(b1, k * _I, (k + 1) * _I, axis=0)
            hr = pltpu.roll(h, _I - 1, 1)
            p = h * jax.nn.sigmoid(1.702 * h) * (hr + 1.0)
            return jax.lax.dot_general(p, sel_ref[...], (((1,), (0,)), ((), ())),
                                       preferred_element_type=jnp.float32)

        s = jnp.concatenate([shalf(w1bufa, 0), shalf(w1bufb, 1)], axis=1)
        ya = jax.lax.dot_general(s, w2bufa[slot], (((1,), (1,)), ((), ())),
                                 preferred_element_type=jnp.float32)
        yb = jax.lax.dot_general(s, w2bufb[slot], (((1,), (1,)), ((), ())),
                                 preferred_element_type=jnp.float32)
        y = jnp.concatenate([ya, yb], axis=1) + b2_ref[0, 0]
        out_ref[...] = y * wp_ref[0, 0][:, None]


def kernel(hidden_states, gate_w, gate_b, w1, b1, w2, b2):
    bsz, seq, hd = hidden_states.shape
    x2 = hidden_states.reshape(-1, hd)

    logits = x2 @ gate_w.T + gate_b
    probs = jax.nn.softmax(logits, axis=-1)
    vals, idx = jax.lax.top_k(probs, _TOPK)
    vals = vals / jnp.sum(vals, axis=-1, keepdims=True)

    e_flat = idx.reshape(-1).astype(jnp.int32)
    v_flat = vals.reshape(-1)
    order = jnp.argsort(e_flat, stable=True)
    rank = jnp.zeros((_R,), jnp.int32).at[order].set(
        jnp.arange(_R, dtype=jnp.int32))
    counts = jnp.bincount(e_flat, length=_E).astype(jnp.int32)
    pcounts = ((counts + _BM - 1) // _BM) * _BM
    pc_cum = jnp.cumsum(pcounts).astype(jnp.int32)
    pstart = pc_cum - pcounts
    g_cum = jnp.cumsum(counts).astype(jnp.int32)
    gstart = g_cum - counts
    total_pad = pc_cum[-1]
    nused = (total_pad // _BM).astype(jnp.int32)
    pos = pstart[e_flat] + (rank - gstart[e_flat])
    src_tok = jnp.zeros((_MAXPAD,), jnp.int32).at[pos].set(
        jnp.arange(_R, dtype=jnp.int32) // _TOPK)
    w_pad = jnp.zeros((_MAXPAD,), jnp.float32).at[pos].set(v_flat)
    queries = (jnp.arange(_NBLK, dtype=jnp.int32) * _BM).astype(jnp.int32)
    be = jnp.searchsorted(pc_cum, queries, side="right").astype(jnp.int32)
    be_last = be[jnp.maximum(nused - 1, 0)]
    be = jnp.where(queries < total_pad, be, be_last)
    info = jnp.concatenate([be, nused[None]])
    fb = jnp.concatenate([jnp.ones((1,), jnp.int32),
                          (be[1:] != be[:-1]).astype(jnp.int32)])
    fb = fb * (queries < total_pad).astype(jnp.int32)
    runid = jnp.cumsum(fb).astype(jnp.int32) - 1
    nruns = jnp.sum(fb).astype(jnp.int32)
    runx = jnp.zeros((_RXN,), jnp.int32).at[runid].set(be)
    runx = runx.at[_RXN - 1].set(nruns)

    x_pad = x2[src_tok]

    b1r = b1.reshape(_E, 1, 2 * _I)
    b2r = b2.reshape(_E, 1, _H)
    wpr = w_pad.reshape(_NBLK, 1, _BM)
    sel = (jnp.arange(_I, dtype=jnp.int32)[:, None]
           == 2 * jnp.arange(_I // 2, dtype=jnp.int32)[None, :]
           ).astype(jnp.float32)
    grid_spec = pltpu.PrefetchScalarGridSpec(
        num_scalar_prefetch=4,
        grid=(_NBLK,),
        in_specs=[
            pl.BlockSpec((_BM, _H), lambda i, *s: (i, 0)),
            pl.BlockSpec(memory_space=pltpu.MemorySpace.HBM),
            pl.BlockSpec((1, 1, 2 * _I), lambda i, *s: (s[0][i], 0, 0)),
            pl.BlockSpec((_I, _I // 2), lambda i, *s: (0, 0)),
            pl.BlockSpec(memory_space=pltpu.MemorySpace.HBM),
            pl.BlockSpec((1, 1, _H), lambda i, *s: (s[0][i], 0, 0)),
            pl.BlockSpec((1, 1, _BM), lambda i, *s: (i, 0, 0)),
        ],
        out_specs=pl.BlockSpec((_BM, _H), lambda i, *s: (i, 0)),
        scratch_shapes=[
            pltpu.VMEM((_NRING, _I, _H), jnp.float32),
            pltpu.VMEM((_NRING, _I, _H), jnp.float32),
            pltpu.VMEM((_NRING, _H // 2, _I), jnp.float32),
            pltpu.VMEM((_NRING, _H // 2, _I), jnp.float32),
            pltpu.SemaphoreType.DMA((_NRING,)),
            pltpu.SemaphoreType.DMA((_NRING,)),
            pltpu.SemaphoreType.DMA((_NRING,)),
            pltpu.SemaphoreType.DMA((_NRING,)),
        ],
    )
    y_pad = pl.pallas_call(
        _ffn_block,
        grid_spec=grid_spec,
        out_shape=jax.ShapeDtypeStruct((_MAXPAD, _H), jnp.float32),
        compiler_params=pltpu.CompilerParams(
            dimension_semantics=("arbitrary",)),
        interpret=_INTERPRET,
    )(info, runid, fb, runx, x_pad, w1, b1r, sel, w2, b2r, wpr)

    p2 = pos.reshape(_T, _TOPK)
    out2 = y_pad[p2[:, 0]] + y_pad[p2[:, 1]]
    return out2.reshape(bsz, seq, hd)


# speedup vs baseline: 1.0270x; 1.0219x over previous
import jax
import jax.numpy as jnp
from jax.experimental import pallas as pl
from jax.experimental.pallas import tpu as pltpu

_E = 64
_TOPK = 2
_H = 768
_I = 768
_T = 2048
_R = _T * _TOPK
_BM = 128
_MAXPAD = _R + _E * _BM
_NBLK = _MAXPAD // _BM
_LA = 3
_NRING = 5
_RXN = _NBLK + _LA + 1

_INTERPRET = False


def _ffn_block(info_ref, runid_ref, fb_ref, runx_ref,
               x_ref, w1_hbm, b1_ref, sel_ref, w2_hbm, b2_ref, wp_ref,
               out_ref, w1bufa, w1bufb, w2bufa, w2bufb,
               sem1a, sem1b, sem2a, sem2b):
    i = pl.program_id(0)
    nused = info_ref[_NBLK]
    nruns = runx_ref[_RXN - 1]

    def copies(run, slot):
        e = runx_ref[run]
        return (
            pltpu.make_async_copy(w1_hbm.at[e, pl.ds(0, _I)],
                                  w1bufa.at[slot], sem1a.at[slot]),
            pltpu.make_async_copy(w1_hbm.at[e, pl.ds(_I, _I)],
                                  w1bufb.at[slot], sem1b.at[slot]),
            pltpu.make_async_copy(w2_hbm.at[e, pl.ds(0, _H // 2)],
                                  w2bufa.at[slot], sem2a.at[slot]),
            pltpu.make_async_copy(w2_hbm.at[e, pl.ds(_H // 2, _H // 2)],
                                  w2bufb.at[slot], sem2b.at[slot]),
        )

    @pl.when(i == 0)
    def _():
        for k in range(_LA):
            @pl.when(k < nruns)
            def _():
                for c in copies(k, k):
                    c.start()

    @pl.when((fb_ref[i] == 1) & (i < nused))
    def _():
        r = runid_ref[i]

        @pl.when(r + _LA < nruns)
        def _():
            for c in copies(r + _LA, jax.lax.rem(r + _LA, _NRING)):
                c.start()

        for c in copies(r, jax.lax.rem(r, _NRING)):
            c.wait()

    @pl.when(i < nused)
    def _():
        slot = jax.lax.rem(runid_ref[i], _NRING)
        x = x_ref[...]
        b1 = b1_ref[0, 0]

        def shalf(wbuf, k):
            h = jax.lax.dot_general(x, wbuf[slot], (((1,), (1,)), ((), ())),
                                    preferred_element_type=jnp.float32)
            h = h + jax.lax.slice_in_dim(b1, k * _I, (k + 1) * _I, axis=0)
            hr = pltpu.roll(h, _I - 1, 1)
            p = h * jax.nn.sigmoid(1.702 * h) * (hr + 1.0)
            return jax.lax.dot_general(p, sel_ref[...], (((1,), (0,)), ((), ())),
                                       preferred_element_type=jnp.float32)

        s = jnp.concatenate([shalf(w1bufa, 0), shalf(w1bufb, 1)], axis=1)
        ya = jax.lax.dot_general(s, w2bufa[slot], (((1,), (1,)), ((), ())),
                                 preferred_element_type=jnp.float32)
        yb = jax.lax.dot_general(s, w2bufb[slot], (((1,), (1,)), ((), ())),
                                 preferred_element_type=jnp.float32)
        y = jnp.concatenate([ya, yb], axis=1) + b2_ref[0, 0]
        out_ref[...] = y * wp_ref[0, 0][:, None]


def kernel(hidden_states, gate_w, gate_b, w1, b1, w2, b2):
    bsz, seq, hd = hidden_states.shape
    x2 = hidden_states.reshape(-1, hd)

    logits = x2 @ gate_w.T + gate_b
    probs = jax.nn.softmax(logits, axis=-1)
    vals, idx = jax.lax.top_k(probs, _TOPK)
    vals = vals / jnp.sum(vals, axis=-1, keepdims=True)

    e_flat = idx.reshape(-1).astype(jnp.int32)
    v_flat = vals.reshape(-1)
    order = jnp.argsort(e_flat, stable=True)
    rank = jnp.zeros((_R,), jnp.int32).at[order].set(
        jnp.arange(_R, dtype=jnp.int32))
    counts = jnp.bincount(e_flat, length=_E).astype(jnp.int32)
    pcounts = ((counts + _BM - 1) // _BM) * _BM
    pc_cum = jnp.cumsum(pcounts).astype(jnp.int32)
    pstart = pc_cum - pcounts
    g_cum = jnp.cumsum(counts).astype(jnp.int32)
    gstart = g_cum - counts
    total_pad = pc_cum[-1]
    nused = (total_pad // _BM).astype(jnp.int32)
    pos = pstart[e_flat] + (rank - gstart[e_flat])
    src_tok = jnp.zeros((_MAXPAD,), jnp.int32).at[pos].set(
        jnp.arange(_R, dtype=jnp.int32) // _TOPK)
    w_pad = jnp.zeros((_MAXPAD,), jnp.float32).at[pos].set(v_flat)
    queries = (jnp.arange(_NBLK, dtype=jnp.int32) * _BM).astype(jnp.int32)
    be = jnp.searchsorted(pc_cum, queries, side="right").astype(jnp.int32)
    be_last = be[jnp.maximum(nused - 1, 0)]
    be = jnp.where(queries < total_pad, be, be_last)
    info = jnp.concatenate([be, nused[None]])
    fb = jnp.concatenate([jnp.ones((1,), jnp.int32),
                          (be[1:] != be[:-1]).astype(jnp.int32)])
    fb = fb * (queries < total_pad).astype(jnp.int32)
    runid = jnp.cumsum(fb).astype(jnp.int32) - 1
    nruns = jnp.sum(fb).astype(jnp.int32)
    runx = jnp.zeros((_RXN,), jnp.int32).at[runid].set(be)
    runx = runx.at[_RXN - 1].set(nruns)

    x_pad = x2[src_tok]

    b1r = b1.reshape(_E, 1, 2 * _I)
    b2r = b2.reshape(_E, 1, _H)
    wpr = w_pad.reshape(_NBLK, 1, _BM)
    sel = (jnp.arange(_I, dtype=jnp.int32)[:, None]
           == 2 * jnp.arange(_I // 2, dtype=jnp.int32)[None, :]
           ).astype(jnp.float32)
    grid_spec = pltpu.PrefetchScalarGridSpec(
        num_scalar_prefetch=4,
        grid=(_NBLK,),
        in_specs=[
            pl.BlockSpec((_BM, _H),
                         lambda i, *s: (jnp.minimum(i, s[0][_NBLK] - 1), 0)),
            pl.BlockSpec(memory_space=pltpu.MemorySpace.HBM),
            pl.BlockSpec((1, 1, 2 * _I), lambda i, *s: (s[0][i], 0, 0)),
            pl.BlockSpec((_I, _I // 2), lambda i, *s: (0, 0)),
            pl.BlockSpec(memory_space=pltpu.MemorySpace.HBM),
            pl.BlockSpec((1, 1, _H), lambda i, *s: (s[0][i], 0, 0)),
            pl.BlockSpec((1, 1, _BM), lambda i, *s: (i, 0, 0)),
        ],
        out_specs=pl.BlockSpec(
            (_BM, _H), lambda i, *s: (jnp.minimum(i, s[0][_NBLK] - 1), 0)),
        scratch_shapes=[
            pltpu.VMEM((_NRING, _I, _H), jnp.float32),
            pltpu.VMEM((_NRING, _I, _H), jnp.float32),
            pltpu.VMEM((_NRING, _H // 2, _I), jnp.float32),
            pltpu.VMEM((_NRING, _H // 2, _I), jnp.float32),
            pltpu.SemaphoreType.DMA((_NRING,)),
            pltpu.SemaphoreType.DMA((_NRING,)),
            pltpu.SemaphoreType.DMA((_NRING,)),
            pltpu.SemaphoreType.DMA((_NRING,)),
        ],
    )
    y_pad = pl.pallas_call(
        _ffn_block,
        grid_spec=grid_spec,
        out_shape=jax.ShapeDtypeStruct((_MAXPAD, _H), jnp.float32),
        compiler_params=pltpu.CompilerParams(
            dimension_semantics=("arbitrary",)),
        interpret=_INTERPRET,
    )(info, runid, fb, runx, x_pad, w1, b1r, sel, w2, b2r, wpr)

    p2 = pos.reshape(_T, _TOPK)
    out2 = y_pad[p2[:, 0]] + y_pad[p2[:, 1]]
    return out2.reshape(bsz, seq, hd)
